# baseline (speedup 1.0000x reference)
import functools

import jax
import jax.numpy as jnp
from jax import lax
from jax.experimental import pallas as pl
from jax.experimental.pallas import tpu as pltpu
from jax.experimental.pallas import tpu_sc as plsc

VOCAB = 1000000
MAX_SEQ = 2048
HID = 64
B = 4096
L = 200

NUM_CORES = 2
NUM_SUBCORES = 16
NUM_WORKERS = NUM_CORES * NUM_SUBCORES
ROWS_PER_W = B // NUM_WORKERS
FLAT_PER_W = ROWS_PER_W * L

CHUNK_A = 104
CHUNK_B = L - CHUNK_A


def _sc_body(x_hbm, tok_hbm, pos_hbm, out_hbm, idx_v, pos_v, rows_v,
             g_sem, s_sem):
    wid = lax.axis_index("s") * NUM_CORES + lax.axis_index("c")
    base = wid * FLAT_PER_W

    pltpu.sync_copy(x_hbm.at[pl.ds(base, FLAT_PER_W)], idx_v)
    pltpu.sync_copy(pos_hbm.at[pl.ds(0, L)], pos_v)

    def row_body(r, _):
        off = r * L
        ca = pltpu.async_copy(
            tok_hbm.at[idx_v.at[pl.ds(off, CHUNK_A)]],
            rows_v.at[pl.ds(0, CHUNK_A)], g_sem)
        cb = pltpu.async_copy(
            tok_hbm.at[idx_v.at[pl.ds(off + CHUNK_A, CHUNK_B)]],
            rows_v.at[pl.ds(CHUNK_A, CHUNK_B)], g_sem)
        ca.wait()
        cb.wait()

        def add_body(l, _):
            for j in range(HID // 16):
                sl = pl.ds(j * 16, 16)
                rows_v[l, sl] = rows_v[l, sl] + pos_v[l, sl]
            return 0
        lax.fori_loop(0, L, add_body, 0, unroll=2)

        pltpu.async_copy(rows_v, out_hbm.at[pl.ds(base + off, L)], s_sem).wait()
        return 0

    lax.fori_loop(0, ROWS_PER_W, row_body, 0)


@jax.jit
def _tpe(x_flat, tok_table, pos_table):
    mesh = plsc.VectorSubcoreMesh(core_axis_name="c", subcore_axis_name="s")
    kern = functools.partial(
        pl.kernel,
        mesh=mesh,
        out_type=jax.ShapeDtypeStruct((B * L, HID), jnp.float32),
        scratch_types=[
            pltpu.VMEM((FLAT_PER_W,), jnp.int32),
            pltpu.VMEM((L, HID), jnp.float32),
            pltpu.VMEM((L, HID), jnp.float32),
            pltpu.SemaphoreType.DMA,
            pltpu.SemaphoreType.DMA,
        ],
        compiler_params=pltpu.CompilerParams(use_tc_tiling_on_sc=False),
    )(_sc_body)
    return kern(x_flat, tok_table, pos_table)


def kernel(x, tok_table, pos_table):
    x_flat = x.reshape(B * L).astype(jnp.int32)
    out = _tpe(x_flat, tok_table, pos_table)
    return out.reshape(B, L, HID)


# speedup vs baseline: 1.2893x; 1.2893x over previous
import functools

import jax
import jax.numpy as jnp
from jax import lax
from jax.experimental import pallas as pl
from jax.experimental.pallas import tpu as pltpu
from jax.experimental.pallas import tpu_sc as plsc

VOCAB = 1000000
MAX_SEQ = 2048
HID = 64
B = 4096
L = 200

NUM_CORES = 2
NUM_SUBCORES = 16
NUM_WORKERS = NUM_CORES * NUM_SUBCORES
ROWS_PER_W = B // NUM_WORKERS
FLAT_PER_W = ROWS_PER_W * L

CHUNK_A = 104
CHUNK_B = L - CHUNK_A

NSLOT = 4
PAIRS = ROWS_PER_W // NSLOT


def _sc_body(x_hbm, tok_hbm, pos_hbm, out_hbm, idx_v, pos_v, rows_v,
             g0, g1, g2, g3, s0, s1, s2, s3):
    g_sems = (g0, g1, g2, g3)
    s_sems = (s0, s1, s2, s3)
    wid = lax.axis_index("s") * NUM_CORES + lax.axis_index("c")
    base = wid * FLAT_PER_W

    pltpu.sync_copy(x_hbm.at[pl.ds(base, FLAT_PER_W)], idx_v)
    pltpu.sync_copy(pos_hbm.at[pl.ds(0, L)], pos_v)

    def issue_gather(r, slot):
        off = r * L
        pltpu.async_copy(
            tok_hbm.at[idx_v.at[pl.ds(off, CHUNK_A)]],
            rows_v.at[slot].at[pl.ds(0, CHUNK_A)], g_sems[slot])
        pltpu.async_copy(
            tok_hbm.at[idx_v.at[pl.ds(off + CHUNK_A, CHUNK_B)]],
            rows_v.at[slot].at[pl.ds(CHUNK_A, CHUNK_B)], g_sems[slot])

    def wait_gather(r, slot):
        pltpu.make_async_copy(
            tok_hbm.at[idx_v.at[pl.ds(0, CHUNK_A)]],
            rows_v.at[slot].at[pl.ds(0, CHUNK_A)], g_sems[slot]).wait()
        pltpu.make_async_copy(
            tok_hbm.at[idx_v.at[pl.ds(0, CHUNK_B)]],
            rows_v.at[slot].at[pl.ds(CHUNK_A, CHUNK_B)], g_sems[slot]).wait()

    def issue_store(r, slot):
        pltpu.async_copy(rows_v.at[slot], out_hbm.at[pl.ds(base + r * L, L)],
                         s_sems[slot])

    def wait_store(slot):
        pltpu.make_async_copy(rows_v.at[slot],
                              out_hbm.at[pl.ds(0, L)], s_sems[slot]).wait()

    def add_pos(slot):
        def add_body(l, _):
            for j in range(HID // 16):
                sl = pl.ds(j * 16, 16)
                rows_v[slot, l, sl] = rows_v[slot, l, sl] + pos_v[l, sl]
            return 0
        lax.fori_loop(0, L, add_body, 0, unroll=4)

    issue_gather(0, 0)
    issue_gather(1, 1)

    def macro_body(i, _):
        r0 = i * NSLOT
        for p in range(NSLOT):
            r = r0 + p
            slot = p
            ahead_slot = (p + 2) % NSLOT

            wait_gather(r, slot)
            if p < 2:
                @pl.when(i > 0)
                def _():
                    wait_store(ahead_slot)
                issue_gather(r + 2, ahead_slot)
            else:
                @pl.when(r + 2 < ROWS_PER_W)
                def _():
                    wait_store(ahead_slot)
                    issue_gather(r + 2, ahead_slot)
            add_pos(slot)
            issue_store(r, slot)
        return 0

    lax.fori_loop(0, PAIRS, macro_body, 0)

    for slot in range(NSLOT):
        wait_store(slot)


@jax.jit
def _tpe(x_flat, tok_table, pos_table):
    mesh = plsc.VectorSubcoreMesh(core_axis_name="c", subcore_axis_name="s")
    kern = functools.partial(
        pl.kernel,
        mesh=mesh,
        out_type=jax.ShapeDtypeStruct((B * L, HID), jnp.float32),
        scratch_types=[
            pltpu.VMEM((FLAT_PER_W,), jnp.int32),
            pltpu.VMEM((L, HID), jnp.float32),
            pltpu.VMEM((NSLOT, L, HID), jnp.float32),
            pltpu.SemaphoreType.DMA,
            pltpu.SemaphoreType.DMA,
            pltpu.SemaphoreType.DMA,
            pltpu.SemaphoreType.DMA,
            pltpu.SemaphoreType.DMA,
            pltpu.SemaphoreType.DMA,
            pltpu.SemaphoreType.DMA,
            pltpu.SemaphoreType.DMA,
        ],
        compiler_params=pltpu.CompilerParams(use_tc_tiling_on_sc=False),
    )(_sc_body)
    return kern(x_flat, tok_table, pos_table)


def kernel(x, tok_table, pos_table):
    x_flat = x.reshape(B * L).astype(jnp.int32)
    out = _tpe(x_flat, tok_table, pos_table)
    return out.reshape(B, L, HID)
